# baseline (speedup 1.0000x reference)
import functools

import jax
import jax.numpy as jnp
from jax.experimental import pallas as pl
from jax.experimental.pallas import tpu as pltpu


def _pos_emb(max_pos, hidden):
    inv_freq = 1.0 / (10000.0 ** (jnp.arange(0, hidden, 2, dtype=jnp.float32) / hidden))
    position = jnp.arange(max_pos, dtype=jnp.float32)
    sinusoid = position[:, None] * inv_freq[None, :]
    return jnp.concatenate([jnp.sin(sinusoid), jnp.cos(sinusoid)], axis=-1)


def _const_kernel(eq_ref, wt_ref, b_ref, o_ref):
    o_ref[...] = (
        jnp.dot(eq_ref[...], wt_ref[...], preferred_element_type=jnp.float32)
        + b_ref[...]
    )


def _sentw_kernel(s_ref, wt_ref, o_ref):
    o_ref[...] = jnp.dot(s_ref[...], wt_ref[...], preferred_element_type=jnp.float32)


def _main_kernel(mask_ref, a_ref, sw_ref, wt_ref, c_ref, o_ref, *, bb, nl, q):
    out_dim = o_ref.shape[-1]
    lens = jnp.sum(mask_ref[...], axis=1, keepdims=True).astype(jnp.int32)
    row = jax.lax.broadcasted_iota(jnp.int32, (q, out_dim), 0)
    c = c_ref[...]
    zeros_tail = jnp.zeros((q - 1 - nl, out_dim), jnp.float32)
    for j in range(bb):
        a = a_ref[j].astype(jnp.bfloat16)
        y = jax.lax.dot_general(
            a, wt_ref[...], (((0,), (0,)), ((), ())),
            preferred_element_type=jnp.float32,
        )
        y0 = sw_ref[j]
        y64 = jnp.concatenate([y0, y, zeros_tail], axis=0)
        ln = lens[j : j + 1, 0:1]
        ow = (row >= 1) & (row <= nl) & (row - 1 >= ln)
        o_ref[j] = jnp.where(ow, y0, y64) + c


def kernel(lang_feat, lang_sent, lang_mask, learnable_query, proj_w, proj_b):
    b, d, nl = lang_feat.shape
    out_dim = proj_w.shape[0]
    learn = learnable_query.shape[0]
    q = 1 + nl + learn

    wt = proj_w.T
    wtb = wt.astype(jnp.bfloat16)
    emb = _pos_emb(q, d)
    embq = emb.at[1 + nl :].add(learnable_query)
    bias2 = proj_b[None, :]
    mask2 = lang_mask[..., 0]
    sent3 = lang_sent[:, None, :]

    const = pl.pallas_call(
        _const_kernel,
        out_shape=jax.ShapeDtypeStruct((q, out_dim), jnp.float32),
    )(embq, wt, bias2)

    sb = min(256, b)
    sentw = pl.pallas_call(
        _sentw_kernel,
        grid=(b // sb,),
        in_specs=[
            pl.BlockSpec((sb, d), lambda i: (i, 0)),
            pl.BlockSpec((d, out_dim), lambda i: (0, 0)),
        ],
        out_specs=pl.BlockSpec((sb, out_dim), lambda i: (i, 0)),
        out_shape=jax.ShapeDtypeStruct((b, out_dim), jnp.float32),
        compiler_params=pltpu.CompilerParams(
            dimension_semantics=("parallel",),
        ),
    )(lang_sent, wt)
    sentw3 = sentw[:, None, :]

    bb = 8
    out = pl.pallas_call(
        functools.partial(_main_kernel, bb=bb, nl=nl, q=q),
        grid=(b // bb,),
        in_specs=[
            pl.BlockSpec((bb, nl), lambda i: (i, 0)),
            pl.BlockSpec((bb, d, nl), lambda i: (i, 0, 0)),
            pl.BlockSpec((bb, 1, out_dim), lambda i: (i, 0, 0)),
            pl.BlockSpec((d, out_dim), lambda i: (0, 0)),
            pl.BlockSpec((q, out_dim), lambda i: (0, 0)),
        ],
        out_specs=pl.BlockSpec((bb, q, out_dim), lambda i: (i, 0, 0)),
        out_shape=jax.ShapeDtypeStruct((b, q, out_dim), jnp.float32),
        compiler_params=pltpu.CompilerParams(
            dimension_semantics=("parallel",),
        ),
    )(mask2, lang_feat, sentw3, wtb, const)
    return out


# speedup vs baseline: 1.3110x; 1.3110x over previous
import functools

import jax
import jax.numpy as jnp
from jax.experimental import pallas as pl
from jax.experimental.pallas import tpu as pltpu


def _pos_emb(max_pos, hidden):
    inv_freq = 1.0 / (10000.0 ** (jnp.arange(0, hidden, 2, dtype=jnp.float32) / hidden))
    position = jnp.arange(max_pos, dtype=jnp.float32)
    sinusoid = position[:, None] * inv_freq[None, :]
    return jnp.concatenate([jnp.sin(sinusoid), jnp.cos(sinusoid)], axis=-1)


def _const_kernel(eq_ref, wt_ref, b_ref, o_ref):
    o_ref[...] = (
        jnp.dot(eq_ref[...], wt_ref[...], preferred_element_type=jnp.float32)
        + b_ref[...]
    )


def _sentw_kernel(s_ref, wt_ref, o_ref):
    o_ref[...] = jnp.dot(s_ref[...], wt_ref[...], preferred_element_type=jnp.float32)


def _main_kernel(mask_ref, a_ref, sw_ref, wt_ref, c_ref, o_ref, *, bb, nl, q):
    out_dim = o_ref.shape[-1]
    lens = jnp.sum(mask_ref[...], axis=1, keepdims=True).astype(jnp.int32)
    row = jax.lax.broadcasted_iota(jnp.int32, (q, out_dim), 0)
    c = c_ref[...]
    zeros_tail = jnp.zeros((q - 1 - nl, out_dim), jnp.float32)
    for j in range(bb):
        a = a_ref[j]
        y = jnp.dot(a, wt_ref[...], preferred_element_type=jnp.float32)
        y0 = sw_ref[j]
        y64 = jnp.concatenate([y0, y, zeros_tail], axis=0)
        ln = lens[j : j + 1, 0:1]
        ow = (row >= 1) & (row <= nl) & (row - 1 >= ln)
        o_ref[j] = jnp.where(ow, y0, y64) + c


def kernel(lang_feat, lang_sent, lang_mask, learnable_query, proj_w, proj_b):
    b, d, nl = lang_feat.shape
    out_dim = proj_w.shape[0]
    learn = learnable_query.shape[0]
    q = 1 + nl + learn

    wt = proj_w.T
    wtb = wt.astype(jnp.bfloat16)
    emb = _pos_emb(q, d)
    embq = emb.at[1 + nl :].add(learnable_query)
    bias2 = proj_b[None, :]
    mask2 = lang_mask[..., 0]
    af = jnp.swapaxes(lang_feat, 1, 2).astype(jnp.bfloat16)

    const = pl.pallas_call(
        _const_kernel,
        out_shape=jax.ShapeDtypeStruct((q, out_dim), jnp.float32),
    )(embq, wt, bias2)

    sb = min(256, b)
    sentw = pl.pallas_call(
        _sentw_kernel,
        grid=(b // sb,),
        in_specs=[
            pl.BlockSpec((sb, d), lambda i: (i, 0)),
            pl.BlockSpec((d, out_dim), lambda i: (0, 0)),
        ],
        out_specs=pl.BlockSpec((sb, out_dim), lambda i: (i, 0)),
        out_shape=jax.ShapeDtypeStruct((b, out_dim), jnp.float32),
        compiler_params=pltpu.CompilerParams(
            dimension_semantics=("parallel",),
        ),
    )(lang_sent, wt)
    sentw3 = sentw[:, None, :]

    bb = 8
    out = pl.pallas_call(
        functools.partial(_main_kernel, bb=bb, nl=nl, q=q),
        grid=(b // bb,),
        in_specs=[
            pl.BlockSpec((bb, nl), lambda i: (i, 0)),
            pl.BlockSpec((bb, nl, d), lambda i: (i, 0, 0)),
            pl.BlockSpec((bb, 1, out_dim), lambda i: (i, 0, 0)),
            pl.BlockSpec((d, out_dim), lambda i: (0, 0)),
            pl.BlockSpec((q, out_dim), lambda i: (0, 0)),
        ],
        out_specs=pl.BlockSpec((bb, q, out_dim), lambda i: (i, 0, 0)),
        out_shape=jax.ShapeDtypeStruct((b, q, out_dim), jnp.float32),
        compiler_params=pltpu.CompilerParams(
            dimension_semantics=("parallel",),
        ),
    )(mask2, af, sentw3, wtb, const)
    return out


# speedup vs baseline: 1.3161x; 1.0039x over previous
import functools

import jax
import jax.numpy as jnp
from jax.experimental import pallas as pl
from jax.experimental.pallas import tpu as pltpu


def _pos_emb(max_pos, hidden):
    inv_freq = 1.0 / (10000.0 ** (jnp.arange(0, hidden, 2, dtype=jnp.float32) / hidden))
    position = jnp.arange(max_pos, dtype=jnp.float32)
    sinusoid = position[:, None] * inv_freq[None, :]
    return jnp.concatenate([jnp.sin(sinusoid), jnp.cos(sinusoid)], axis=-1)


def _const_kernel(eq_ref, wt_ref, b_ref, o_ref):
    o_ref[...] = (
        jnp.dot(eq_ref[...], wt_ref[...], preferred_element_type=jnp.float32)
        + b_ref[...]
    )


def _sentw_kernel(s_ref, wt_ref, o_ref):
    o_ref[...] = jnp.dot(s_ref[...], wt_ref[...], preferred_element_type=jnp.float32)


def _main_kernel(mask_ref, a_ref, sw_ref, wt_ref, c_ref, o_ref, *, bb, nl, q):
    out_dim = o_ref.shape[-1]
    lens = jnp.sum(mask_ref[...], axis=1, keepdims=True).astype(jnp.int32)
    row = jax.lax.broadcasted_iota(jnp.int32, (q, out_dim), 0)
    c = c_ref[...]
    zeros_tail = jnp.zeros((q - 1 - nl, out_dim), jnp.float32)
    for j in range(bb):
        a = a_ref[j]
        y = jnp.dot(a, wt_ref[...], preferred_element_type=jnp.float32)
        y0 = sw_ref[j]
        y64 = jnp.concatenate([y0, y, zeros_tail], axis=0)
        ln = lens[j : j + 1, 0:1]
        ow = (row >= 1) & (row <= nl) & (row - 1 >= ln)
        o_ref[j] = jnp.where(ow, y0, y64) + c


def kernel(lang_feat, lang_sent, lang_mask, learnable_query, proj_w, proj_b):
    b, d, nl = lang_feat.shape
    out_dim = proj_w.shape[0]
    learn = learnable_query.shape[0]
    q = 1 + nl + learn

    wt = proj_w.T
    wtb = wt.astype(jnp.bfloat16)
    emb = _pos_emb(q, d)
    embq = emb.at[1 + nl :].add(learnable_query)
    bias2 = proj_b[None, :]
    mask2 = lang_mask[..., 0]
    af = jnp.swapaxes(lang_feat, 1, 2).astype(jnp.bfloat16)

    const = pl.pallas_call(
        _const_kernel,
        out_shape=jax.ShapeDtypeStruct((q, out_dim), jnp.float32),
    )(embq, wt, bias2)

    sb = min(256, b)
    sentw = pl.pallas_call(
        _sentw_kernel,
        grid=(b // sb,),
        in_specs=[
            pl.BlockSpec((sb, d), lambda i: (i, 0)),
            pl.BlockSpec((d, out_dim), lambda i: (0, 0)),
        ],
        out_specs=pl.BlockSpec((sb, out_dim), lambda i: (i, 0)),
        out_shape=jax.ShapeDtypeStruct((b, out_dim), jnp.float32),
        compiler_params=pltpu.CompilerParams(
            dimension_semantics=("parallel",),
        ),
    )(lang_sent, wt)
    sentw3 = sentw[:, None, :]

    bb = 16
    out = pl.pallas_call(
        functools.partial(_main_kernel, bb=bb, nl=nl, q=q),
        grid=(b // bb,),
        in_specs=[
            pl.BlockSpec((bb, nl), lambda i: (i, 0)),
            pl.BlockSpec((bb, nl, d), lambda i: (i, 0, 0)),
            pl.BlockSpec((bb, 1, out_dim), lambda i: (i, 0, 0)),
            pl.BlockSpec((d, out_dim), lambda i: (0, 0)),
            pl.BlockSpec((q, out_dim), lambda i: (0, 0)),
        ],
        out_specs=pl.BlockSpec((bb, q, out_dim), lambda i: (i, 0, 0)),
        out_shape=jax.ShapeDtypeStruct((b, q, out_dim), jnp.float32),
        compiler_params=pltpu.CompilerParams(
            dimension_semantics=("parallel",),
        ),
    )(mask2, af, sentw3, wtb, const)
    return out


# speedup vs baseline: 1.8512x; 1.4066x over previous
import functools

import jax
import jax.numpy as jnp
from jax.experimental import pallas as pl
from jax.experimental.pallas import tpu as pltpu


def _pos_emb(max_pos, hidden):
    inv_freq = 1.0 / (10000.0 ** (jnp.arange(0, hidden, 2, dtype=jnp.float32) / hidden))
    position = jnp.arange(max_pos, dtype=jnp.float32)
    sinusoid = position[:, None] * inv_freq[None, :]
    return jnp.concatenate([jnp.sin(sinusoid), jnp.cos(sinusoid)], axis=-1)


def _const_kernel(eq_ref, wt_ref, b_ref, o_ref):
    o_ref[...] = (
        jnp.dot(eq_ref[...], wt_ref[...], preferred_element_type=jnp.float32)
        + b_ref[...]
    )


def _sentw_kernel(s_ref, wt_ref, o_ref):
    o_ref[...] = jnp.dot(s_ref[...], wt_ref[...], preferred_element_type=jnp.float32)


def _main_kernel(mask_ref, a_ref, sw_ref, wt_ref, c_ref, o_ref, *, bb, nl, q):
    out_dim = o_ref.shape[-1]
    rows = bb * nl
    lens = jnp.sum(mask_ref[...], axis=1, keepdims=True).astype(jnp.int32)
    lens_rows = jnp.broadcast_to(lens[:, None, :], (bb, nl, 1)).reshape(rows, 1)
    y = jnp.dot(a_ref[...], wt_ref[...], preferred_element_type=jnp.float32)
    y0_exp = jnp.broadcast_to(sw_ref[...], (bb, nl, out_dim)).reshape(rows, out_dim)
    l_iota = jax.lax.broadcasted_iota(jnp.int32, (rows, 1), 0) % nl
    ysel = jnp.where(l_iota >= lens_rows, y0_exp, y)
    c = c_ref[...]
    zeros_tail = jnp.zeros((q - 1 - nl, out_dim), jnp.float32)
    for j in range(bb):
        y64 = jnp.concatenate(
            [sw_ref[j], ysel[j * nl : (j + 1) * nl], zeros_tail], axis=0
        )
        o_ref[j] = y64 + c


def kernel(lang_feat, lang_sent, lang_mask, learnable_query, proj_w, proj_b):
    b, d, nl = lang_feat.shape
    out_dim = proj_w.shape[0]
    learn = learnable_query.shape[0]
    q = 1 + nl + learn

    wt = proj_w.T
    wtb = wt.astype(jnp.bfloat16)
    emb = _pos_emb(q, d)
    embq = emb.at[1 + nl :].add(learnable_query)
    bias2 = proj_b[None, :]
    mask2 = lang_mask[..., 0]
    af = jnp.swapaxes(lang_feat, 1, 2).astype(jnp.bfloat16).reshape(b * nl, d)

    const = pl.pallas_call(
        _const_kernel,
        out_shape=jax.ShapeDtypeStruct((q, out_dim), jnp.float32),
    )(embq, wt, bias2)

    sb = min(256, b)
    sentw = pl.pallas_call(
        _sentw_kernel,
        grid=(b // sb,),
        in_specs=[
            pl.BlockSpec((sb, d), lambda i: (i, 0)),
            pl.BlockSpec((d, out_dim), lambda i: (0, 0)),
        ],
        out_specs=pl.BlockSpec((sb, out_dim), lambda i: (i, 0)),
        out_shape=jax.ShapeDtypeStruct((b, out_dim), jnp.float32),
        compiler_params=pltpu.CompilerParams(
            dimension_semantics=("parallel",),
        ),
    )(lang_sent, wt)
    sentw3 = sentw[:, None, :]

    bb = 16
    out = pl.pallas_call(
        functools.partial(_main_kernel, bb=bb, nl=nl, q=q),
        grid=(b // bb,),
        in_specs=[
            pl.BlockSpec((bb, nl), lambda i: (i, 0)),
            pl.BlockSpec((bb * nl, d), lambda i: (i, 0)),
            pl.BlockSpec((bb, 1, out_dim), lambda i: (i, 0, 0)),
            pl.BlockSpec((d, out_dim), lambda i: (0, 0)),
            pl.BlockSpec((q, out_dim), lambda i: (0, 0)),
        ],
        out_specs=pl.BlockSpec((bb, q, out_dim), lambda i: (i, 0, 0)),
        out_shape=jax.ShapeDtypeStruct((b, q, out_dim), jnp.float32),
        compiler_params=pltpu.CompilerParams(
            dimension_semantics=("parallel",),
        ),
    )(mask2, af, sentw3, wtb, const)
    return out


# speedup vs baseline: 1.8766x; 1.0137x over previous
import functools

import jax
import jax.numpy as jnp
from jax.experimental import pallas as pl
from jax.experimental.pallas import tpu as pltpu


def _pos_emb(max_pos, hidden):
    inv_freq = 1.0 / (10000.0 ** (jnp.arange(0, hidden, 2, dtype=jnp.float32) / hidden))
    position = jnp.arange(max_pos, dtype=jnp.float32)
    sinusoid = position[:, None] * inv_freq[None, :]
    return jnp.concatenate([jnp.sin(sinusoid), jnp.cos(sinusoid)], axis=-1)


def _const_kernel(eq_ref, wt_ref, b_ref, o_ref):
    o_ref[...] = (
        jnp.dot(eq_ref[...], wt_ref[...], preferred_element_type=jnp.float32)
        + b_ref[...]
    )


def _sentw_kernel(s_ref, wt_ref, o_ref):
    y = jnp.dot(
        s_ref[...].astype(jnp.bfloat16),
        wt_ref[...],
        preferred_element_type=jnp.float32,
    )
    o_ref[...] = y[:, None, :]


def _main_kernel(mask_ref, a_ref, sw_ref, wt_ref, c_ref, o_ref, *, bb, nl, q):
    out_dim = o_ref.shape[-1]
    rows = bb * nl
    lens = jnp.sum(mask_ref[...], axis=1, keepdims=True).astype(jnp.int32)
    lens_rows = jnp.broadcast_to(lens[:, None, :], (bb, nl, 1)).reshape(rows, 1)
    y = jnp.dot(a_ref[...], wt_ref[...], preferred_element_type=jnp.float32)
    y0_exp = jnp.broadcast_to(sw_ref[...], (bb, nl, out_dim)).reshape(rows, out_dim)
    l_iota = jax.lax.broadcasted_iota(jnp.int32, (rows, 1), 0) % nl
    ysel = jnp.where(l_iota >= lens_rows, y0_exp, y)
    c = c_ref[...]
    zeros_tail = jnp.zeros((q - 1 - nl, out_dim), jnp.float32)
    for j in range(bb):
        y64 = jnp.concatenate(
            [sw_ref[j], ysel[j * nl : (j + 1) * nl], zeros_tail], axis=0
        )
        o_ref[j] = y64 + c


def kernel(lang_feat, lang_sent, lang_mask, learnable_query, proj_w, proj_b):
    b, d, nl = lang_feat.shape
    out_dim = proj_w.shape[0]
    learn = learnable_query.shape[0]
    q = 1 + nl + learn

    wtb = proj_w.T.astype(jnp.bfloat16)
    emb = _pos_emb(q, d)
    embq = (emb.at[1 + nl :].add(learnable_query)).astype(jnp.bfloat16)
    bias2 = proj_b[None, :]
    mask2 = lang_mask[..., 0]
    af = jnp.swapaxes(lang_feat, 1, 2).astype(jnp.bfloat16).reshape(b * nl, d)

    const = pl.pallas_call(
        _const_kernel,
        out_shape=jax.ShapeDtypeStruct((q, out_dim), jnp.float32),
    )(embq, wtb, bias2)

    sb = min(256, b)
    sentw3 = pl.pallas_call(
        _sentw_kernel,
        grid=(b // sb,),
        in_specs=[
            pl.BlockSpec((sb, d), lambda i: (i, 0)),
            pl.BlockSpec((d, out_dim), lambda i: (0, 0)),
        ],
        out_specs=pl.BlockSpec((sb, 1, out_dim), lambda i: (i, 0, 0)),
        out_shape=jax.ShapeDtypeStruct((b, 1, out_dim), jnp.float32),
        compiler_params=pltpu.CompilerParams(
            dimension_semantics=("parallel",),
        ),
    )(lang_sent, wtb)

    bb = 16
    out = pl.pallas_call(
        functools.partial(_main_kernel, bb=bb, nl=nl, q=q),
        grid=(b // bb,),
        in_specs=[
            pl.BlockSpec((bb, nl), lambda i: (i, 0)),
            pl.BlockSpec((bb * nl, d), lambda i: (i, 0)),
            pl.BlockSpec((bb, 1, out_dim), lambda i: (i, 0, 0)),
            pl.BlockSpec((d, out_dim), lambda i: (0, 0)),
            pl.BlockSpec((q, out_dim), lambda i: (0, 0)),
        ],
        out_specs=pl.BlockSpec((bb, q, out_dim), lambda i: (i, 0, 0)),
        out_shape=jax.ShapeDtypeStruct((b, q, out_dim), jnp.float32),
        compiler_params=pltpu.CompilerParams(
            dimension_semantics=("parallel",),
        ),
    )(mask2, af, sentw3, wtb, const)
    return out


# speedup vs baseline: 1.8820x; 1.0029x over previous
import functools

import jax
import jax.numpy as jnp
from jax import lax
from jax.experimental import pallas as pl
from jax.experimental.pallas import tpu as pltpu
from jax.experimental.pallas import tpu_sc as plsc


def _pos_emb(max_pos, hidden):
    inv_freq = 1.0 / (10000.0 ** (jnp.arange(0, hidden, 2, dtype=jnp.float32) / hidden))
    position = jnp.arange(max_pos, dtype=jnp.float32)
    sinusoid = position[:, None] * inv_freq[None, :]
    return jnp.concatenate([jnp.sin(sinusoid), jnp.cos(sinusoid)], axis=-1)


def _lengths_sc(nl, nw, nc, bpw):
    def body(maskT_hbm, out_hbm, v_mask, v_acc):
        wid = lax.axis_index("s") * nc + lax.axis_index("c")
        pltpu.sync_copy(maskT_hbm.at[:, wid], v_mask)
        for h in range(bpw // 16):
            acc = jnp.zeros((16,), jnp.float32)
            for l in range(nl):
                acc = acc + v_mask[l, h * 16 : (h + 1) * 16]
            v_acc[h * 16 : (h + 1) * 16] = acc
        pltpu.sync_copy(v_acc, out_hbm.at[pl.ds(wid * bpw, bpw)])

    return body


def _const_kernel(eq_ref, wt_ref, b_ref, o_ref):
    o_ref[...] = (
        jnp.dot(eq_ref[...], wt_ref[...], preferred_element_type=jnp.float32)
        + b_ref[...]
    )


def _sentw_kernel(s_ref, wt_ref, o_ref):
    y = jnp.dot(
        s_ref[...].astype(jnp.bfloat16),
        wt_ref[...],
        preferred_element_type=jnp.float32,
    )
    o_ref[...] = y[:, None, :]


def _main_kernel(lens_ref, a_ref, sw_ref, wt_ref, c_ref, o_ref, *, bb, nl, q):
    out_dim = o_ref.shape[-1]
    rows = bb * nl
    lens = lens_ref[...].astype(jnp.int32)
    lens_rows = jnp.broadcast_to(lens[:, None, :], (bb, nl, 1)).reshape(rows, 1)
    y = jnp.dot(a_ref[...], wt_ref[...], preferred_element_type=jnp.float32)
    y0_exp = jnp.broadcast_to(sw_ref[...], (bb, nl, out_dim)).reshape(rows, out_dim)
    l_iota = jax.lax.broadcasted_iota(jnp.int32, (rows, 1), 0) % nl
    ysel = jnp.where(l_iota >= lens_rows, y0_exp, y)
    c = c_ref[...]
    zeros_tail = jnp.zeros((q - 1 - nl, out_dim), jnp.float32)
    for j in range(bb):
        y64 = jnp.concatenate(
            [sw_ref[j], ysel[j * nl : (j + 1) * nl], zeros_tail], axis=0
        )
        o_ref[j] = y64 + c


def kernel(lang_feat, lang_sent, lang_mask, learnable_query, proj_w, proj_b):
    b, d, nl = lang_feat.shape
    out_dim = proj_w.shape[0]
    learn = learnable_query.shape[0]
    q = 1 + nl + learn

    wtb = proj_w.T.astype(jnp.bfloat16)
    emb = _pos_emb(q, d)
    embq = (emb.at[1 + nl :].add(learnable_query)).astype(jnp.bfloat16)
    bias2 = proj_b[None, :]
    af = jnp.swapaxes(lang_feat, 1, 2).astype(jnp.bfloat16).reshape(b * nl, d)

    info = plsc.get_sparse_core_info()
    nc, ns = info.num_cores, info.num_subcores
    nw = nc * ns
    bpw = b // nw
    maskT3 = jnp.transpose(lang_mask[..., 0]).reshape(nl, nw, bpw)
    lens_fn = functools.partial(
        pl.kernel,
        mesh=plsc.VectorSubcoreMesh(core_axis_name="c", subcore_axis_name="s"),
        out_type=jax.ShapeDtypeStruct((b,), jnp.float32),
        scratch_types=[
            pltpu.VMEM((nl, bpw), jnp.float32),
            pltpu.VMEM((bpw,), jnp.float32),
        ],
    )(_lengths_sc(nl, nw, nc, bpw))
    lens2 = lens_fn(maskT3).reshape(b, 1)

    const = pl.pallas_call(
        _const_kernel,
        out_shape=jax.ShapeDtypeStruct((q, out_dim), jnp.float32),
    )(embq, wtb, bias2)

    sb = min(256, b)
    sentw3 = pl.pallas_call(
        _sentw_kernel,
        grid=(b // sb,),
        in_specs=[
            pl.BlockSpec((sb, d), lambda i: (i, 0)),
            pl.BlockSpec((d, out_dim), lambda i: (0, 0)),
        ],
        out_specs=pl.BlockSpec((sb, 1, out_dim), lambda i: (i, 0, 0)),
        out_shape=jax.ShapeDtypeStruct((b, 1, out_dim), jnp.float32),
        compiler_params=pltpu.CompilerParams(
            dimension_semantics=("parallel",),
        ),
    )(lang_sent, wtb)

    bb = 16
    out = pl.pallas_call(
        functools.partial(_main_kernel, bb=bb, nl=nl, q=q),
        grid=(b // bb,),
        in_specs=[
            pl.BlockSpec((bb, 1), lambda i: (i, 0)),
            pl.BlockSpec((bb * nl, d), lambda i: (i, 0)),
            pl.BlockSpec((bb, 1, out_dim), lambda i: (i, 0, 0)),
            pl.BlockSpec((d, out_dim), lambda i: (0, 0)),
            pl.BlockSpec((q, out_dim), lambda i: (0, 0)),
        ],
        out_specs=pl.BlockSpec((bb, q, out_dim), lambda i: (i, 0, 0)),
        out_shape=jax.ShapeDtypeStruct((b, q, out_dim), jnp.float32),
        compiler_params=pltpu.CompilerParams(
            dimension_semantics=("parallel",),
        ),
    )(lens2, af, sentw3, wtb, const)
    return out
